# baseline (speedup 1.0000x reference)
import functools

import jax
import jax.numpy as jnp
from jax import lax
from jax.experimental import pallas as pl
from jax.experimental.pallas import tpu as pltpu
from jax.experimental.pallas import tpu_sc as plsc

N = 10000
BN = 512
NP = 10752
GN = NP // BN
E = 320000
C = 64
NW = 32
CPW = 160
ECH = NW * CPW
EP = ECH * C
IB = 8
NIB = CPW // IB
BE = 4096
GE = EP // BE
ACC_R = 10112
RPT = ACC_R // 16
H = 64
EPS = 1e-7


def _node_mm_body(x_ref, w_ref, b_ref, h_ref, mx_ref):
    i = pl.program_id(0)
    h = jnp.dot(x_ref[...], w_ref[...], preferred_element_type=jnp.float32)
    h = h + b_ref[...]
    h_ref[...] = jnp.concatenate(
        [h, jnp.zeros((BN, H), jnp.float32)], axis=1)
    bm = jnp.full((8, 128), jnp.max(h), dtype=jnp.float32)

    @pl.when(i == 0)
    def _():
        mx_ref[...] = bm

    @pl.when(i != 0)
    def _():
        mx_ref[...] = jnp.maximum(mx_ref[...], bm)


_node_mm = pl.pallas_call(
    _node_mm_body,
    grid=(GN,),
    in_specs=[
        pl.BlockSpec((BN, 128), lambda i: (i, 0)),
        pl.BlockSpec((128, H), lambda i: (0, 0)),
        pl.BlockSpec((1, H), lambda i: (0, 0)),
    ],
    out_specs=[
        pl.BlockSpec((BN, 128), lambda i: (i, 0)),
        pl.BlockSpec((8, 128), lambda i: (0, 0)),
    ],
    out_shape=[
        jax.ShapeDtypeStruct((NP, 128), jnp.float32),
        jax.ShapeDtypeStruct((8, 128), jnp.float32),
    ],
)


def _edge_mlp_body(a_ref, w1_ref, b1_ref, w2_ref, b2_ref, ea_ref, mx_ref):
    i = pl.program_id(0)
    u = jnp.dot(a_ref[...], w1_ref[...], preferred_element_type=jnp.float32)
    u = jnp.maximum(u + b1_ref[...], 0.0)
    e = jnp.dot(u, w2_ref[...], preferred_element_type=jnp.float32)
    e = e + b2_ref[...]
    ea_ref[...] = e
    bm = jnp.full((8, 128), jnp.max(e), dtype=jnp.float32)

    @pl.when(i == 0)
    def _():
        mx_ref[...] = bm

    @pl.when(i != 0)
    def _():
        mx_ref[...] = jnp.maximum(mx_ref[...], bm)


_edge_mlp = pl.pallas_call(
    _edge_mlp_body,
    grid=(GE,),
    in_specs=[
        pl.BlockSpec((BE, 16), lambda i: (i, 0)),
        pl.BlockSpec((16, 32), lambda i: (0, 0)),
        pl.BlockSpec((1, 32), lambda i: (0, 0)),
        pl.BlockSpec((32, H), lambda i: (0, 0)),
        pl.BlockSpec((1, H), lambda i: (0, 0)),
    ],
    out_specs=[
        pl.BlockSpec((BE, H), lambda i: (i, 0)),
        pl.BlockSpec((8, 128), lambda i: (0, 0)),
    ],
    out_shape=[
        jax.ShapeDtypeStruct((EP, H), jnp.float32),
        jax.ShapeDtypeStruct((8, 128), jnp.float32),
    ],
)


_SC_MESH = plsc.VectorSubcoreMesh(
    core_axis_name="c", subcore_axis_name="s", num_cores=2, num_subcores=16
)


def _sc_edge_body(hn_hbm, ea_hbm, sd_hbm, tv_hbm, mv_hbm, zr_hbm, out_hbm,
                  sd0, sd1, ea0, ea1, g0, g1, wm0, wm1, tv_v, mv_v, acc,
                  sg0, sg1, se0, se1, sw0, sw1, si0, si1):
    core = lax.axis_index("c")
    tile = lax.axis_index("s")
    w = core * 16 + tile
    cbase = w * CPW
    sd = (sd0, sd1)
    eab = (ea0, ea1)
    gb = (g0, g1)
    wmb = (wm0, wm1)
    sg = (sg0, sg1)
    se = (se0, se1)
    sw = (sw0, sw1)
    si = (si0, si1)
    pltpu.sync_copy(zr_hbm, acc.at[pl.ds(tile * RPT, RPT)])
    pltpu.sync_copy(tv_hbm, tv_v)
    pltpu.sync_copy(mv_hbm, mv_v)
    tvec = tv_v[...]
    mvec = mv_v[...]
    plsc.subcore_barrier()

    pltpu.sync_copy(sd_hbm.at[pl.ds(cbase, IB)], sd0)
    pltpu.async_copy(sd_hbm.at[pl.ds(cbase + IB, IB)], sd1, si1)
    for kk in (0, 1):
        pltpu.async_copy(
            ea_hbm.at[pl.ds((cbase + kk) * (C // 2), C // 2)], eab[kk],
            se[kk])
        pltpu.async_copy(hn_hbm.at[sd0.at[kk, 0]], gb[kk], sg[kk])

    def pair_body(j, sj, p):
        for kk in (0, 1):
            k = IB * j + 2 * p + kk
            pltpu.make_async_copy(ea_hbm.at[pl.ds(0, C // 2)], eab[kk],
                                  se[kk]).wait()
            pltpu.make_async_copy(zr_hbm.at[pl.ds(0, C)], gb[kk],
                                  sg[kk]).wait()

            @pl.when(k >= 2)
            def _():
                pltpu.make_async_copy(zr_hbm.at[pl.ds(0, C)], wmb[kk],
                                      sw[kk]).wait()

            def edge2(e2, c2):
                for sub in (0, 1):
                    e = 2 * e2 + sub
                    for cc in range(4):
                        sl = pl.ds(cc * 16, 16)
                        eav = eab[kk][e2, pl.ds(sub * 64 + cc * 16, 16)]
                        msg = jnp.maximum(gb[kk][e, sl] + eav, 0.0) + EPS
                        wgt = jnp.exp(msg * tvec - mvec)
                        wmb[kk][e, sl] = wgt
                        wmb[kk][e, pl.ds(H + cc * 16, 16)] = msg * wgt
                return c2

            lax.fori_loop(0, C // 2, edge2, 0)
            pltpu.async_copy(wmb[kk], acc.at[sd[sj].at[2 * p + kk, 1]],
                             sw[kk], add=True)

            @pl.when(p < IB // 2 - 1)
            def _():
                m = k + 2
                pltpu.async_copy(
                    ea_hbm.at[pl.ds((cbase + m) * (C // 2), C // 2)],
                    eab[kk], se[kk])
                pltpu.async_copy(hn_hbm.at[sd[sj].at[2 * p + kk + 2, 0]],
                                 gb[kk], sg[kk])

            @pl.when((p == IB // 2 - 1) & (j + 1 < NIB))
            def _():
                if kk == 0:
                    pltpu.make_async_copy(
                        sd_hbm.at[pl.ds(cbase, IB)], sd[1 - sj],
                        si[1 - sj]).wait()
                m = IB * (j + 1) + kk
                pltpu.async_copy(
                    ea_hbm.at[pl.ds((cbase + m) * (C // 2), C // 2)],
                    eab[kk], se[kk])
                pltpu.async_copy(hn_hbm.at[sd[1 - sj].at[kk, 0]], gb[kk],
                                 sg[kk])

            if kk == 1:
                @pl.when((p == 0) & (j >= 1) & (j + 1 < NIB))
                def _():
                    pltpu.async_copy(
                        sd_hbm.at[pl.ds(cbase + (j + 1) * IB, IB)],
                        sd[1 - sj], si[1 - sj])

    def blkpair(J, carry):
        for sj in (0, 1):
            j = 2 * J + sj

            def pair(p, c2):
                pair_body(j, sj, p)
                return c2

            lax.fori_loop(0, IB // 2, pair, 0)
        return carry

    lax.fori_loop(0, NIB // 2, blkpair, 0)

    pltpu.make_async_copy(zr_hbm.at[pl.ds(0, C)], wm0, sw0).wait()
    pltpu.make_async_copy(zr_hbm.at[pl.ds(0, C)], wm1, sw1).wait()
    plsc.subcore_barrier()
    pltpu.sync_copy(acc.at[pl.ds(tile * RPT, RPT)],
                    out_hbm.at[core, pl.ds(tile * RPT, RPT)])


_sc_edge = pl.kernel(
    _sc_edge_body,
    out_type=jax.ShapeDtypeStruct((2, NP, 2 * H), jnp.float32),
    mesh=_SC_MESH,
    scratch_types=[
        pltpu.VMEM((IB, 2, C), jnp.int32),
        pltpu.VMEM((IB, 2, C), jnp.int32),
        pltpu.VMEM((C // 2, 128), jnp.float32),
        pltpu.VMEM((C // 2, 128), jnp.float32),
        pltpu.VMEM((C, 2 * H), jnp.float32),
        pltpu.VMEM((C, 2 * H), jnp.float32),
        pltpu.VMEM((C, 2 * H), jnp.float32),
        pltpu.VMEM((C, 2 * H), jnp.float32),
        pltpu.VMEM((16,), jnp.float32),
        pltpu.VMEM((16,), jnp.float32),
        pltpu.VMEM_SHARED((ACC_R, 2 * H), jnp.float32),
        pltpu.SemaphoreType.DMA,
        pltpu.SemaphoreType.DMA,
        pltpu.SemaphoreType.DMA,
        pltpu.SemaphoreType.DMA,
        pltpu.SemaphoreType.DMA,
        pltpu.SemaphoreType.DMA,
        pltpu.SemaphoreType.DMA,
        pltpu.SemaphoreType.DMA,
    ],
)


def _post_mid_body(dnm_ref, hn_ref, hres_ref, w1_ref, b1_ref, g1_ref, bb1_ref,
                   w2_ref, b2_ref, lng_ref, lnb_ref, h_ref, hn2_ref, mx_ref):
    i = pl.program_id(0)
    p = dnm_ref[0] + dnm_ref[1]
    aggr = p[:, H:] / (p[:, :H] + 1e-16)
    u = jnp.dot(aggr + hn_ref[:, :H], w1_ref[...],
                preferred_element_type=jnp.float32) + b1_ref[...]
    m = jnp.mean(u, axis=-1, keepdims=True)
    v = jnp.mean((u - m) ** 2, axis=-1, keepdims=True)
    u = (u - m) / jnp.sqrt(v + 1e-5) * g1_ref[...] + bb1_ref[...]
    u = jnp.maximum(u, 0.0)
    hnew = jnp.dot(u, w2_ref[...],
                   preferred_element_type=jnp.float32) + b2_ref[...]
    hnew = hnew + hres_ref[...]
    h_ref[...] = hnew
    m2 = jnp.mean(hnew, axis=-1, keepdims=True)
    v2 = jnp.mean((hnew - m2) ** 2, axis=-1, keepdims=True)
    hn2 = jnp.maximum(
        (hnew - m2) / jnp.sqrt(v2 + 1e-5) * lng_ref[...] + lnb_ref[...], 0.0)
    hn2_ref[...] = jnp.concatenate(
        [hn2, jnp.zeros((BN, H), jnp.float32)], axis=1)
    rows = lax.broadcasted_iota(jnp.int32, (BN, H), 0) + i * BN
    bm = jnp.full((8, 128), jnp.max(jnp.where(rows < N, hn2, 0.0)),
                  dtype=jnp.float32)

    @pl.when(i == 0)
    def _():
        mx_ref[...] = bm

    @pl.when(i != 0)
    def _():
        mx_ref[...] = jnp.maximum(mx_ref[...], bm)


_post_mid = pl.pallas_call(
    _post_mid_body,
    grid=(GN,),
    in_specs=[
        pl.BlockSpec((2, BN, 2 * H), lambda i: (0, i, 0)),
        pl.BlockSpec((BN, 128), lambda i: (i, 0)),
        pl.BlockSpec((BN, H), lambda i: (i, 0)),
        pl.BlockSpec((H, 2 * H), lambda i: (0, 0)),
        pl.BlockSpec((1, 2 * H), lambda i: (0, 0)),
        pl.BlockSpec((1, 2 * H), lambda i: (0, 0)),
        pl.BlockSpec((1, 2 * H), lambda i: (0, 0)),
        pl.BlockSpec((2 * H, H), lambda i: (0, 0)),
        pl.BlockSpec((1, H), lambda i: (0, 0)),
        pl.BlockSpec((1, H), lambda i: (0, 0)),
        pl.BlockSpec((1, H), lambda i: (0, 0)),
    ],
    out_specs=[
        pl.BlockSpec((BN, H), lambda i: (i, 0)),
        pl.BlockSpec((BN, 128), lambda i: (i, 0)),
        pl.BlockSpec((8, 128), lambda i: (0, 0)),
    ],
    out_shape=[
        jax.ShapeDtypeStruct((NP, H), jnp.float32),
        jax.ShapeDtypeStruct((NP, 128), jnp.float32),
        jax.ShapeDtypeStruct((8, 128), jnp.float32),
    ],
)


def _post_last_body(dnm_ref, hn_ref, hres_ref, w1_ref, b1_ref, g1_ref, bb1_ref,
                    w2_ref, b2_ref, lng_ref, lnb_ref, ow_ref, ob_ref,
                    out_ref, pool_ref):
    i = pl.program_id(0)
    p = dnm_ref[0] + dnm_ref[1]
    aggr = p[:, H:] / (p[:, :H] + 1e-16)
    u = jnp.dot(aggr + hn_ref[:, :H], w1_ref[...],
                preferred_element_type=jnp.float32) + b1_ref[...]
    m = jnp.mean(u, axis=-1, keepdims=True)
    v = jnp.mean((u - m) ** 2, axis=-1, keepdims=True)
    u = (u - m) / jnp.sqrt(v + 1e-5) * g1_ref[...] + bb1_ref[...]
    u = jnp.maximum(u, 0.0)
    hnew = jnp.dot(u, w2_ref[...],
                   preferred_element_type=jnp.float32) + b2_ref[...]
    hnew = hnew + hres_ref[...]
    m2 = jnp.mean(hnew, axis=-1, keepdims=True)
    v2 = jnp.mean((hnew - m2) ** 2, axis=-1, keepdims=True)
    hf = jnp.maximum(
        (hnew - m2) / jnp.sqrt(v2 + 1e-5) * lng_ref[...] + lnb_ref[...], 0.0)
    rows = lax.broadcasted_iota(jnp.int32, (BN, H), 0) + i * BN
    hf = jnp.where(rows < N, hf, 0.0)
    bm = jnp.max(hf, axis=0, keepdims=True)

    @pl.when(i == 0)
    def _():
        pool_ref[...] = bm

    @pl.when(i != 0)
    def _():
        pool_ref[...] = jnp.maximum(pool_ref[...], bm)

    @pl.when(i == GN - 1)
    def _():
        s = jnp.sum(pool_ref[...] * ow_ref[...]) + ob_ref[...]
        out_ref[...] = 1.0 / (1.0 + jnp.exp(-s))


_post_last = pl.pallas_call(
    _post_last_body,
    grid=(GN,),
    in_specs=[
        pl.BlockSpec((2, BN, 2 * H), lambda i: (0, i, 0)),
        pl.BlockSpec((BN, 128), lambda i: (i, 0)),
        pl.BlockSpec((BN, H), lambda i: (i, 0)),
        pl.BlockSpec((H, 2 * H), lambda i: (0, 0)),
        pl.BlockSpec((1, 2 * H), lambda i: (0, 0)),
        pl.BlockSpec((1, 2 * H), lambda i: (0, 0)),
        pl.BlockSpec((1, 2 * H), lambda i: (0, 0)),
        pl.BlockSpec((2 * H, H), lambda i: (0, 0)),
        pl.BlockSpec((1, H), lambda i: (0, 0)),
        pl.BlockSpec((1, H), lambda i: (0, 0)),
        pl.BlockSpec((1, H), lambda i: (0, 0)),
        pl.BlockSpec((1, H), lambda i: (0, 0)),
        pl.BlockSpec((1, 1), lambda i: (0, 0)),
    ],
    out_specs=pl.BlockSpec((1, 1), lambda i: (0, 0)),
    out_shape=jax.ShapeDtypeStruct((1, 1), jnp.float32),
    scratch_shapes=[pltpu.VMEM((1, H), jnp.float32)],
)


def kernel(x, edge_index, edge_attr, batch, node_W, node_b, e1_W, e1_b, e2_W,
           e2_b, ln_g, ln_b, mlp1_W, mlp1_b, mln_g, mln_b, mlp2_W, mlp2_b, t,
           out_W, out_b):
    f32 = jnp.float32
    xp = jnp.pad(x, ((0, NP - N), (0, 0)))
    src2 = jnp.concatenate(
        [edge_index[0], jnp.zeros((EP - E,), jnp.int32)]).reshape(ECH, C)
    dst2 = jnp.concatenate(
        [edge_index[1], jnp.full((EP - E,), N, jnp.int32)]).reshape(ECH, C)
    sd = jnp.stack([src2, dst2], axis=1)
    eap = jnp.pad(edge_attr, ((0, EP - E), (0, 0)))
    zr = jnp.zeros((RPT, 2 * H), f32)

    h0, mxh0 = _node_mm(xp, node_W, node_b.reshape(1, H))
    ea, mxe = _edge_mlp(eap, e1_W, e1_b.reshape(1, 32), e2_W,
                        e2_b.reshape(1, H))
    ea = ea.reshape(EP // 2, 128)
    maxea = jnp.max(mxe)

    hn = h0
    hres = jnp.zeros((NP, H), f32)
    maxh = jnp.max(mxh0)
    for i in range(3):
        ti = t[i]
        msgmax = jnp.maximum(maxh + maxea, 0.0) + EPS
        Mi = jnp.where(ti >= 0, msgmax * ti, EPS * ti)
        tv = jnp.full((16,), ti, f32)
        mv = jnp.full((16,), Mi, f32)
        dnm = _sc_edge(hn, ea, sd, tv, mv, zr)
        if i < 2:
            hres, hn, mxh = _post_mid(
                dnm, hn, hres, mlp1_W[i], mlp1_b[i].reshape(1, 2 * H),
                mln_g[i].reshape(1, 2 * H), mln_b[i].reshape(1, 2 * H),
                mlp2_W[i], mlp2_b[i].reshape(1, H),
                ln_g[i + 1].reshape(1, H), ln_b[i + 1].reshape(1, H))
            maxh = jnp.max(mxh)
        else:
            out = _post_last(
                dnm, hn, hres, mlp1_W[i], mlp1_b[i].reshape(1, 2 * H),
                mln_g[i].reshape(1, 2 * H), mln_b[i].reshape(1, 2 * H),
                mlp2_W[i], mlp2_b[i].reshape(1, H),
                ln_g[0].reshape(1, H), ln_b[0].reshape(1, H),
                out_W.reshape(1, H), out_b.reshape(1, 1))
    return out


# speedup vs baseline: 1.3967x; 1.3967x over previous
import jax
import jax.numpy as jnp
from jax import lax
from jax.experimental import pallas as pl
from jax.experimental.pallas import tpu as pltpu
from jax.experimental.pallas import tpu_sc as plsc

N = 10000
BN = 512
NP = 10752
GN = NP // BN
E = 320000
C = 32
NW = 32
CH_TOT = 10240
EP = CH_TOT * C
IB = 32
CPW0 = 448
CPW1 = 192
NIB0 = CPW0 // IB
NIB1 = CPW1 // IB
BE = 4096
GE = EP // BE
ACC_R = 10112
RPT = ACC_R // 16
H = 64
EPS = 1e-7


def _node_mm_body(x_ref, w_ref, b_ref, h_ref, mx_ref):
    i = pl.program_id(0)
    h = jnp.dot(x_ref[...], w_ref[...], preferred_element_type=jnp.float32)
    h = h + b_ref[...]
    h_ref[...] = jnp.concatenate(
        [h, jnp.zeros((BN, H), jnp.float32)], axis=1)
    bm = jnp.full((8, 128), jnp.max(h), dtype=jnp.float32)

    @pl.when(i == 0)
    def _():
        mx_ref[...] = bm

    @pl.when(i != 0)
    def _():
        mx_ref[...] = jnp.maximum(mx_ref[...], bm)


_node_mm = pl.pallas_call(
    _node_mm_body,
    grid=(GN,),
    in_specs=[
        pl.BlockSpec((BN, 128), lambda i: (i, 0)),
        pl.BlockSpec((128, H), lambda i: (0, 0)),
        pl.BlockSpec((1, H), lambda i: (0, 0)),
    ],
    out_specs=[
        pl.BlockSpec((BN, 128), lambda i: (i, 0)),
        pl.BlockSpec((8, 128), lambda i: (0, 0)),
    ],
    out_shape=[
        jax.ShapeDtypeStruct((NP, 128), jnp.float32),
        jax.ShapeDtypeStruct((8, 128), jnp.float32),
    ],
)


def _edge_mlp_body(a_ref, w1_ref, b1_ref, w2_ref, b2_ref, ea_ref, mx_ref):
    i = pl.program_id(0)
    u = jnp.dot(a_ref[...], w1_ref[...], preferred_element_type=jnp.float32)
    u = jnp.maximum(u + b1_ref[...], 0.0)
    e = jnp.dot(u, w2_ref[...], preferred_element_type=jnp.float32)
    e = e + b2_ref[...]
    ea_ref[...] = e
    bm = jnp.full((8, 128), jnp.max(e), dtype=jnp.float32)

    @pl.when(i == 0)
    def _():
        mx_ref[...] = bm

    @pl.when(i != 0)
    def _():
        mx_ref[...] = jnp.maximum(mx_ref[...], bm)


_edge_mlp = pl.pallas_call(
    _edge_mlp_body,
    grid=(GE,),
    in_specs=[
        pl.BlockSpec((BE, 16), lambda i: (i, 0)),
        pl.BlockSpec((16, 32), lambda i: (0, 0)),
        pl.BlockSpec((1, 32), lambda i: (0, 0)),
        pl.BlockSpec((32, H), lambda i: (0, 0)),
        pl.BlockSpec((1, H), lambda i: (0, 0)),
    ],
    out_specs=[
        pl.BlockSpec((BE, H), lambda i: (i, 0)),
        pl.BlockSpec((8, 128), lambda i: (0, 0)),
    ],
    out_shape=[
        jax.ShapeDtypeStruct((EP, H), jnp.float32),
        jax.ShapeDtypeStruct((8, 128), jnp.float32),
    ],
)


_SC_MESH = plsc.VectorSubcoreMesh(
    core_axis_name="c", subcore_axis_name="s", num_cores=2, num_subcores=16
)


def _sc_edge_body(hn_hbm, ea_hbm, sd_hbm, tv_hbm, mv_hbm, zr_hbm, out_hbm,
                  sd0, sd1, ea0, ea1, g0, g1, wm0, wm1, tv_v, mv_v, acc,
                  sg0, sg1, se0, se1, sw0, sw1, si0, si1):
    core = lax.axis_index("c")
    tile = lax.axis_index("s")
    cbase = lax.select(core == 0, tile * CPW0, 16 * CPW0 + tile * CPW1)
    nib = lax.select(core == 0, NIB0, NIB1)
    sd = (sd0, sd1)
    eab = (ea0, ea1)
    gb = (g0, g1)
    wmb = (wm0, wm1)
    sg = (sg0, sg1)
    se = (se0, se1)
    sw = (sw0, sw1)
    si = (si0, si1)
    pltpu.sync_copy(zr_hbm, acc.at[pl.ds(tile * RPT, RPT)])
    pltpu.sync_copy(tv_hbm, tv_v)
    pltpu.sync_copy(mv_hbm, mv_v)
    tvec = tv_v[...]
    mvec = mv_v[...]
    plsc.subcore_barrier()

    pltpu.sync_copy(sd_hbm.at[pl.ds(cbase, IB)], sd0)
    pltpu.async_copy(sd_hbm.at[pl.ds(cbase + IB, IB)], sd1, si1)
    for kk in (0, 1):
        pltpu.async_copy(ea_hbm.at[pl.ds((cbase + kk) * C, C)], eab[kk],
                         se[kk])
        pltpu.async_copy(hn_hbm.at[sd0.at[kk, 0]], gb[kk], sg[kk])

    def pair_body(j, sj, p):
        for kk in (0, 1):
            k = IB * j + 2 * p + kk
            pltpu.make_async_copy(ea_hbm.at[pl.ds(0, C)], eab[kk],
                                  se[kk]).wait()
            pltpu.make_async_copy(zr_hbm.at[pl.ds(0, C)], gb[kk],
                                  sg[kk]).wait()

            @pl.when(k >= 2)
            def _():
                pltpu.make_async_copy(zr_hbm.at[pl.ds(0, C)], wmb[kk],
                                      sw[kk]).wait()

            def edge(e, c2):
                for cc in range(4):
                    sl = pl.ds(cc * 16, 16)
                    msg = jnp.maximum(gb[kk][e, sl] + eab[kk][e, sl],
                                      0.0) + EPS
                    wgt = jnp.exp(msg * tvec - mvec)
                    wmb[kk][e, sl] = wgt
                    wmb[kk][e, pl.ds(H + cc * 16, 16)] = msg * wgt
                return c2

            lax.fori_loop(0, C, edge, 0)
            pltpu.async_copy(wmb[kk], acc.at[sd[sj].at[2 * p + kk, 1]],
                             sw[kk], add=True)

            @pl.when(p < IB // 2 - 1)
            def _():
                m = k + 2
                pltpu.async_copy(ea_hbm.at[pl.ds((cbase + m) * C, C)],
                                 eab[kk], se[kk])
                pltpu.async_copy(hn_hbm.at[sd[sj].at[2 * p + kk + 2, 0]],
                                 gb[kk], sg[kk])

            @pl.when((p == IB // 2 - 1) & (j + 1 < nib))
            def _():
                if kk == 0:
                    pltpu.make_async_copy(
                        sd_hbm.at[pl.ds(cbase, IB)], sd[1 - sj],
                        si[1 - sj]).wait()
                m = IB * (j + 1) + kk
                pltpu.async_copy(ea_hbm.at[pl.ds((cbase + m) * C, C)],
                                 eab[kk], se[kk])
                pltpu.async_copy(hn_hbm.at[sd[1 - sj].at[kk, 0]], gb[kk],
                                 sg[kk])

            if kk == 1:
                @pl.when((p == 0) & (j >= 1) & (j + 1 < nib))
                def _():
                    pltpu.async_copy(
                        sd_hbm.at[pl.ds(cbase + (j + 1) * IB, IB)],
                        sd[1 - sj], si[1 - sj])

    def blkpair(J, carry):
        for sj in (0, 1):
            j = 2 * J + sj

            def pair(p, c2):
                pair_body(j, sj, p)
                return c2

            lax.fori_loop(0, IB // 2, pair, 0)
        return carry

    lax.fori_loop(0, lax.select(core == 0, NIB0 // 2, NIB1 // 2), blkpair, 0)

    pltpu.make_async_copy(zr_hbm.at[pl.ds(0, C)], wm0, sw0).wait()
    pltpu.make_async_copy(zr_hbm.at[pl.ds(0, C)], wm1, sw1).wait()
    plsc.subcore_barrier()
    pltpu.sync_copy(acc.at[pl.ds(tile * RPT, RPT)],
                    out_hbm.at[core, pl.ds(tile * RPT, RPT)])


_sc_edge = pl.kernel(
    _sc_edge_body,
    out_type=jax.ShapeDtypeStruct((2, NP, 2 * H), jnp.float32),
    mesh=_SC_MESH,
    scratch_types=[
        pltpu.VMEM((IB, 2, C), jnp.int32),
        pltpu.VMEM((IB, 2, C), jnp.int32),
        pltpu.VMEM((C, H), jnp.float32),
        pltpu.VMEM((C, H), jnp.float32),
        pltpu.VMEM((C, 2 * H), jnp.float32),
        pltpu.VMEM((C, 2 * H), jnp.float32),
        pltpu.VMEM((C, 2 * H), jnp.float32),
        pltpu.VMEM((C, 2 * H), jnp.float32),
        pltpu.VMEM((16,), jnp.float32),
        pltpu.VMEM((16,), jnp.float32),
        pltpu.VMEM_SHARED((ACC_R, 2 * H), jnp.float32),
        pltpu.SemaphoreType.DMA,
        pltpu.SemaphoreType.DMA,
        pltpu.SemaphoreType.DMA,
        pltpu.SemaphoreType.DMA,
        pltpu.SemaphoreType.DMA,
        pltpu.SemaphoreType.DMA,
        pltpu.SemaphoreType.DMA,
        pltpu.SemaphoreType.DMA,
    ],
)


def _post_mid_body(dnm_ref, hn_ref, hres_ref, w1_ref, b1_ref, g1_ref, bb1_ref,
                   w2_ref, b2_ref, lng_ref, lnb_ref, h_ref, hn2_ref, mx_ref):
    i = pl.program_id(0)
    p = dnm_ref[0] + dnm_ref[1]
    aggr = p[:, H:] / (p[:, :H] + 1e-16)
    u = jnp.dot(aggr + hn_ref[:, :H], w1_ref[...],
                preferred_element_type=jnp.float32) + b1_ref[...]
    m = jnp.mean(u, axis=-1, keepdims=True)
    v = jnp.mean((u - m) ** 2, axis=-1, keepdims=True)
    u = (u - m) / jnp.sqrt(v + 1e-5) * g1_ref[...] + bb1_ref[...]
    u = jnp.maximum(u, 0.0)
    hnew = jnp.dot(u, w2_ref[...],
                   preferred_element_type=jnp.float32) + b2_ref[...]
    hnew = hnew + hres_ref[...]
    h_ref[...] = hnew
    m2 = jnp.mean(hnew, axis=-1, keepdims=True)
    v2 = jnp.mean((hnew - m2) ** 2, axis=-1, keepdims=True)
    hn2 = jnp.maximum(
        (hnew - m2) / jnp.sqrt(v2 + 1e-5) * lng_ref[...] + lnb_ref[...], 0.0)
    hn2_ref[...] = jnp.concatenate(
        [hn2, jnp.zeros((BN, H), jnp.float32)], axis=1)
    rows = lax.broadcasted_iota(jnp.int32, (BN, H), 0) + i * BN
    bm = jnp.full((8, 128), jnp.max(jnp.where(rows < N, hn2, 0.0)),
                  dtype=jnp.float32)

    @pl.when(i == 0)
    def _():
        mx_ref[...] = bm

    @pl.when(i != 0)
    def _():
        mx_ref[...] = jnp.maximum(mx_ref[...], bm)


_post_mid = pl.pallas_call(
    _post_mid_body,
    grid=(GN,),
    in_specs=[
        pl.BlockSpec((2, BN, 2 * H), lambda i: (0, i, 0)),
        pl.BlockSpec((BN, 128), lambda i: (i, 0)),
        pl.BlockSpec((BN, H), lambda i: (i, 0)),
        pl.BlockSpec((H, 2 * H), lambda i: (0, 0)),
        pl.BlockSpec((1, 2 * H), lambda i: (0, 0)),
        pl.BlockSpec((1, 2 * H), lambda i: (0, 0)),
        pl.BlockSpec((1, 2 * H), lambda i: (0, 0)),
        pl.BlockSpec((2 * H, H), lambda i: (0, 0)),
        pl.BlockSpec((1, H), lambda i: (0, 0)),
        pl.BlockSpec((1, H), lambda i: (0, 0)),
        pl.BlockSpec((1, H), lambda i: (0, 0)),
    ],
    out_specs=[
        pl.BlockSpec((BN, H), lambda i: (i, 0)),
        pl.BlockSpec((BN, 128), lambda i: (i, 0)),
        pl.BlockSpec((8, 128), lambda i: (0, 0)),
    ],
    out_shape=[
        jax.ShapeDtypeStruct((NP, H), jnp.float32),
        jax.ShapeDtypeStruct((NP, 128), jnp.float32),
        jax.ShapeDtypeStruct((8, 128), jnp.float32),
    ],
)


def _post_last_body(dnm_ref, hn_ref, hres_ref, w1_ref, b1_ref, g1_ref, bb1_ref,
                    w2_ref, b2_ref, lng_ref, lnb_ref, ow_ref, ob_ref,
                    out_ref, pool_ref):
    i = pl.program_id(0)
    p = dnm_ref[0] + dnm_ref[1]
    aggr = p[:, H:] / (p[:, :H] + 1e-16)
    u = jnp.dot(aggr + hn_ref[:, :H], w1_ref[...],
                preferred_element_type=jnp.float32) + b1_ref[...]
    m = jnp.mean(u, axis=-1, keepdims=True)
    v = jnp.mean((u - m) ** 2, axis=-1, keepdims=True)
    u = (u - m) / jnp.sqrt(v + 1e-5) * g1_ref[...] + bb1_ref[...]
    u = jnp.maximum(u, 0.0)
    hnew = jnp.dot(u, w2_ref[...],
                   preferred_element_type=jnp.float32) + b2_ref[...]
    hnew = hnew + hres_ref[...]
    m2 = jnp.mean(hnew, axis=-1, keepdims=True)
    v2 = jnp.mean((hnew - m2) ** 2, axis=-1, keepdims=True)
    hf = jnp.maximum(
        (hnew - m2) / jnp.sqrt(v2 + 1e-5) * lng_ref[...] + lnb_ref[...], 0.0)
    rows = lax.broadcasted_iota(jnp.int32, (BN, H), 0) + i * BN
    hf = jnp.where(rows < N, hf, 0.0)
    bm = jnp.max(hf, axis=0, keepdims=True)

    @pl.when(i == 0)
    def _():
        pool_ref[...] = bm

    @pl.when(i != 0)
    def _():
        pool_ref[...] = jnp.maximum(pool_ref[...], bm)

    @pl.when(i == GN - 1)
    def _():
        s = jnp.sum(pool_ref[...] * ow_ref[...]) + ob_ref[...]
        out_ref[...] = 1.0 / (1.0 + jnp.exp(-s))


_post_last = pl.pallas_call(
    _post_last_body,
    grid=(GN,),
    in_specs=[
        pl.BlockSpec((2, BN, 2 * H), lambda i: (0, i, 0)),
        pl.BlockSpec((BN, 128), lambda i: (i, 0)),
        pl.BlockSpec((BN, H), lambda i: (i, 0)),
        pl.BlockSpec((H, 2 * H), lambda i: (0, 0)),
        pl.BlockSpec((1, 2 * H), lambda i: (0, 0)),
        pl.BlockSpec((1, 2 * H), lambda i: (0, 0)),
        pl.BlockSpec((1, 2 * H), lambda i: (0, 0)),
        pl.BlockSpec((2 * H, H), lambda i: (0, 0)),
        pl.BlockSpec((1, H), lambda i: (0, 0)),
        pl.BlockSpec((1, H), lambda i: (0, 0)),
        pl.BlockSpec((1, H), lambda i: (0, 0)),
        pl.BlockSpec((1, H), lambda i: (0, 0)),
        pl.BlockSpec((1, 1), lambda i: (0, 0)),
    ],
    out_specs=pl.BlockSpec((1, 1), lambda i: (0, 0)),
    out_shape=jax.ShapeDtypeStruct((1, 1), jnp.float32),
    scratch_shapes=[pltpu.VMEM((1, H), jnp.float32)],
)


def kernel(x, edge_index, edge_attr, batch, node_W, node_b, e1_W, e1_b, e2_W,
           e2_b, ln_g, ln_b, mlp1_W, mlp1_b, mln_g, mln_b, mlp2_W, mlp2_b, t,
           out_W, out_b):
    f32 = jnp.float32
    xp = jnp.pad(x, ((0, NP - N), (0, 0)))
    src2 = jnp.concatenate(
        [edge_index[0], jnp.zeros((EP - E,), jnp.int32)]).reshape(CH_TOT, C)
    dst2 = jnp.concatenate(
        [edge_index[1], jnp.full((EP - E,), N, jnp.int32)]).reshape(CH_TOT, C)
    sd = jnp.stack([src2, dst2], axis=1)
    eap = jnp.pad(edge_attr, ((0, EP - E), (0, 0)))
    zr = jnp.zeros((RPT, 2 * H), f32)

    h0, mxh0 = _node_mm(xp, node_W, node_b.reshape(1, H))
    ea, mxe = _edge_mlp(eap, e1_W, e1_b.reshape(1, 32), e2_W,
                        e2_b.reshape(1, H))
    maxea = jnp.max(mxe)

    hn = h0
    hres = jnp.zeros((NP, H), f32)
    maxh = jnp.max(mxh0)
    for i in range(3):
        ti = t[i]
        msgmax = jnp.maximum(maxh + maxea, 0.0) + EPS
        Mi = jnp.where(ti >= 0, msgmax * ti, EPS * ti)
        tv = jnp.full((16,), ti, f32)
        mv = jnp.full((16,), Mi, f32)
        dnm = _sc_edge(hn, ea, sd, tv, mv, zr)
        if i < 2:
            hres, hn, mxh = _post_mid(
                dnm, hn, hres, mlp1_W[i], mlp1_b[i].reshape(1, 2 * H),
                mln_g[i].reshape(1, 2 * H), mln_b[i].reshape(1, 2 * H),
                mlp2_W[i], mlp2_b[i].reshape(1, H),
                ln_g[i + 1].reshape(1, H), ln_b[i + 1].reshape(1, H))
            maxh = jnp.max(mxh)
        else:
            out = _post_last(
                dnm, hn, hres, mlp1_W[i], mlp1_b[i].reshape(1, 2 * H),
                mln_g[i].reshape(1, 2 * H), mln_b[i].reshape(1, 2 * H),
                mlp2_W[i], mlp2_b[i].reshape(1, H),
                ln_g[0].reshape(1, H), ln_b[0].reshape(1, H),
                out_W.reshape(1, H), out_b.reshape(1, 1))
    return out


# speedup vs baseline: 1.4365x; 1.0285x over previous
import jax
import jax.numpy as jnp
from jax import lax
from jax.experimental import pallas as pl
from jax.experimental.pallas import tpu as pltpu
from jax.experimental.pallas import tpu_sc as plsc

N = 10000
BN = 512
NP = 10752
GN = NP // BN
E = 320000
C = 32
NW = 32
CH_TOT = 10240
EP = CH_TOT * C
IB = 32
CPW0 = 576
CPW1 = 64
NIB0 = CPW0 // IB
NIB1 = CPW1 // IB
BE = 4096
GE = EP // BE
ACC_R = 10112
RPT = ACC_R // 16
H = 64
EPS = 1e-7


def _node_mm_body(x_ref, w_ref, b_ref, h_ref, mx_ref):
    i = pl.program_id(0)
    h = jnp.dot(x_ref[...], w_ref[...], preferred_element_type=jnp.float32)
    h = h + b_ref[...]
    h_ref[...] = jnp.concatenate(
        [h, jnp.zeros((BN, H), jnp.float32)], axis=1)
    bm = jnp.full((8, 128), jnp.max(h), dtype=jnp.float32)

    @pl.when(i == 0)
    def _():
        mx_ref[...] = bm

    @pl.when(i != 0)
    def _():
        mx_ref[...] = jnp.maximum(mx_ref[...], bm)


_node_mm = pl.pallas_call(
    _node_mm_body,
    grid=(GN,),
    in_specs=[
        pl.BlockSpec((BN, 128), lambda i: (i, 0)),
        pl.BlockSpec((128, H), lambda i: (0, 0)),
        pl.BlockSpec((1, H), lambda i: (0, 0)),
    ],
    out_specs=[
        pl.BlockSpec((BN, 128), lambda i: (i, 0)),
        pl.BlockSpec((8, 128), lambda i: (0, 0)),
    ],
    out_shape=[
        jax.ShapeDtypeStruct((NP, 128), jnp.float32),
        jax.ShapeDtypeStruct((8, 128), jnp.float32),
    ],
)


def _edge_mlp_body(a_ref, w1_ref, b1_ref, w2_ref, b2_ref, ea_ref, mx_ref):
    i = pl.program_id(0)
    u = jnp.dot(a_ref[...], w1_ref[...], preferred_element_type=jnp.float32)
    u = jnp.maximum(u + b1_ref[...], 0.0)
    e = jnp.dot(u, w2_ref[...], preferred_element_type=jnp.float32)
    e = e + b2_ref[...]
    ea_ref[...] = e
    bm = jnp.full((8, 128), jnp.max(e), dtype=jnp.float32)

    @pl.when(i == 0)
    def _():
        mx_ref[...] = bm

    @pl.when(i != 0)
    def _():
        mx_ref[...] = jnp.maximum(mx_ref[...], bm)


_edge_mlp = pl.pallas_call(
    _edge_mlp_body,
    grid=(GE,),
    in_specs=[
        pl.BlockSpec((BE, 16), lambda i: (i, 0)),
        pl.BlockSpec((16, 32), lambda i: (0, 0)),
        pl.BlockSpec((1, 32), lambda i: (0, 0)),
        pl.BlockSpec((32, H), lambda i: (0, 0)),
        pl.BlockSpec((1, H), lambda i: (0, 0)),
    ],
    out_specs=[
        pl.BlockSpec((BE, H), lambda i: (i, 0)),
        pl.BlockSpec((8, 128), lambda i: (0, 0)),
    ],
    out_shape=[
        jax.ShapeDtypeStruct((EP, H), jnp.float32),
        jax.ShapeDtypeStruct((8, 128), jnp.float32),
    ],
)


_SC_MESH = plsc.VectorSubcoreMesh(
    core_axis_name="c", subcore_axis_name="s", num_cores=2, num_subcores=16
)


def _sc_edge_body(hn_hbm, ea_hbm, sd_hbm, tv_hbm, mv_hbm, zr_hbm, out_hbm,
                  sd0, sd1, ea0, ea1, g0, g1, wm0, wm1, tv_v, mv_v, acc,
                  sg0, sg1, se0, se1, sw0, sw1, si0, si1):
    core = lax.axis_index("c")
    tile = lax.axis_index("s")
    cbase = lax.select(core == 0, tile * CPW0, 16 * CPW0 + tile * CPW1)
    nib = lax.select(core == 0, NIB0, NIB1)
    sd = (sd0, sd1)
    eab = (ea0, ea1)
    gb = (g0, g1)
    wmb = (wm0, wm1)
    sg = (sg0, sg1)
    se = (se0, se1)
    sw = (sw0, sw1)
    si = (si0, si1)
    pltpu.sync_copy(zr_hbm, acc.at[pl.ds(tile * RPT, RPT)])
    pltpu.sync_copy(tv_hbm, tv_v)
    pltpu.sync_copy(mv_hbm, mv_v)
    tvec = tv_v[...]
    mvec = mv_v[...]
    plsc.subcore_barrier()

    pltpu.sync_copy(sd_hbm.at[pl.ds(cbase, IB)], sd0)
    pltpu.async_copy(sd_hbm.at[pl.ds(cbase + IB, IB)], sd1, si1)
    for kk in (0, 1):
        pltpu.async_copy(ea_hbm.at[pl.ds((cbase + kk) * C, C)], eab[kk],
                         se[kk])
        pltpu.async_copy(hn_hbm.at[sd0.at[kk, 0]], gb[kk], sg[kk])

    def pair_body(j, sj, p):
        for kk in (0, 1):
            k = IB * j + 2 * p + kk
            pltpu.make_async_copy(ea_hbm.at[pl.ds(0, C)], eab[kk],
                                  se[kk]).wait()
            pltpu.make_async_copy(zr_hbm.at[pl.ds(0, C)], gb[kk],
                                  sg[kk]).wait()

            @pl.when(k >= 2)
            def _():
                pltpu.make_async_copy(zr_hbm.at[pl.ds(0, C)], wmb[kk],
                                      sw[kk]).wait()

            def edge(e, c2):
                for cc in range(4):
                    sl = pl.ds(cc * 16, 16)
                    msg = jnp.maximum(gb[kk][e, sl] + eab[kk][e, sl],
                                      0.0) + EPS
                    wgt = jnp.exp(msg * tvec - mvec)
                    wmb[kk][e, sl] = wgt
                    wmb[kk][e, pl.ds(H + cc * 16, 16)] = msg * wgt
                return c2

            lax.fori_loop(0, C, edge, 0)
            pltpu.async_copy(wmb[kk], acc.at[sd[sj].at[2 * p + kk, 1]],
                             sw[kk], add=True)

            @pl.when(p < IB // 2 - 1)
            def _():
                m = k + 2
                pltpu.async_copy(ea_hbm.at[pl.ds((cbase + m) * C, C)],
                                 eab[kk], se[kk])
                pltpu.async_copy(hn_hbm.at[sd[sj].at[2 * p + kk + 2, 0]],
                                 gb[kk], sg[kk])

            @pl.when((p == IB // 2 - 1) & (j + 1 < nib))
            def _():
                if kk == 0:
                    pltpu.make_async_copy(
                        sd_hbm.at[pl.ds(cbase, IB)], sd[1 - sj],
                        si[1 - sj]).wait()
                m = IB * (j + 1) + kk
                pltpu.async_copy(ea_hbm.at[pl.ds((cbase + m) * C, C)],
                                 eab[kk], se[kk])
                pltpu.async_copy(hn_hbm.at[sd[1 - sj].at[kk, 0]], gb[kk],
                                 sg[kk])

            if kk == 1:
                @pl.when((p == 0) & (j >= 1) & (j + 1 < nib))
                def _():
                    pltpu.async_copy(
                        sd_hbm.at[pl.ds(cbase + (j + 1) * IB, IB)],
                        sd[1 - sj], si[1 - sj])

    def blkpair(J, carry):
        for sj in (0, 1):
            j = 2 * J + sj

            def pair(p, c2):
                pair_body(j, sj, p)
                return c2

            lax.fori_loop(0, IB // 2, pair, 0)
        return carry

    lax.fori_loop(0, lax.select(core == 0, NIB0 // 2, NIB1 // 2), blkpair, 0)

    pltpu.make_async_copy(zr_hbm.at[pl.ds(0, C)], wm0, sw0).wait()
    pltpu.make_async_copy(zr_hbm.at[pl.ds(0, C)], wm1, sw1).wait()
    plsc.subcore_barrier()
    pltpu.sync_copy(acc.at[pl.ds(tile * RPT, RPT)],
                    out_hbm.at[core, pl.ds(tile * RPT, RPT)])


_sc_edge = pl.kernel(
    _sc_edge_body,
    out_type=jax.ShapeDtypeStruct((2, NP, 2 * H), jnp.float32),
    mesh=_SC_MESH,
    scratch_types=[
        pltpu.VMEM((IB, 2, C), jnp.int32),
        pltpu.VMEM((IB, 2, C), jnp.int32),
        pltpu.VMEM((C, H), jnp.float32),
        pltpu.VMEM((C, H), jnp.float32),
        pltpu.VMEM((C, 2 * H), jnp.float32),
        pltpu.VMEM((C, 2 * H), jnp.float32),
        pltpu.VMEM((C, 2 * H), jnp.float32),
        pltpu.VMEM((C, 2 * H), jnp.float32),
        pltpu.VMEM((16,), jnp.float32),
        pltpu.VMEM((16,), jnp.float32),
        pltpu.VMEM_SHARED((ACC_R, 2 * H), jnp.float32),
        pltpu.SemaphoreType.DMA,
        pltpu.SemaphoreType.DMA,
        pltpu.SemaphoreType.DMA,
        pltpu.SemaphoreType.DMA,
        pltpu.SemaphoreType.DMA,
        pltpu.SemaphoreType.DMA,
        pltpu.SemaphoreType.DMA,
        pltpu.SemaphoreType.DMA,
    ],
)


def _post_mid_body(dnm_ref, hn_ref, hres_ref, w1_ref, b1_ref, g1_ref, bb1_ref,
                   w2_ref, b2_ref, lng_ref, lnb_ref, h_ref, hn2_ref, mx_ref):
    i = pl.program_id(0)
    p = dnm_ref[0] + dnm_ref[1]
    aggr = p[:, H:] / (p[:, :H] + 1e-16)
    u = jnp.dot(aggr + hn_ref[:, :H], w1_ref[...],
                preferred_element_type=jnp.float32) + b1_ref[...]
    m = jnp.mean(u, axis=-1, keepdims=True)
    v = jnp.mean((u - m) ** 2, axis=-1, keepdims=True)
    u = (u - m) / jnp.sqrt(v + 1e-5) * g1_ref[...] + bb1_ref[...]
    u = jnp.maximum(u, 0.0)
    hnew = jnp.dot(u, w2_ref[...],
                   preferred_element_type=jnp.float32) + b2_ref[...]
    hnew = hnew + hres_ref[...]
    h_ref[...] = hnew
    m2 = jnp.mean(hnew, axis=-1, keepdims=True)
    v2 = jnp.mean((hnew - m2) ** 2, axis=-1, keepdims=True)
    hn2 = jnp.maximum(
        (hnew - m2) / jnp.sqrt(v2 + 1e-5) * lng_ref[...] + lnb_ref[...], 0.0)
    hn2_ref[...] = jnp.concatenate(
        [hn2, jnp.zeros((BN, H), jnp.float32)], axis=1)
    rows = lax.broadcasted_iota(jnp.int32, (BN, H), 0) + i * BN
    bm = jnp.full((8, 128), jnp.max(jnp.where(rows < N, hn2, 0.0)),
                  dtype=jnp.float32)

    @pl.when(i == 0)
    def _():
        mx_ref[...] = bm

    @pl.when(i != 0)
    def _():
        mx_ref[...] = jnp.maximum(mx_ref[...], bm)


_post_mid = pl.pallas_call(
    _post_mid_body,
    grid=(GN,),
    in_specs=[
        pl.BlockSpec((2, BN, 2 * H), lambda i: (0, i, 0)),
        pl.BlockSpec((BN, 128), lambda i: (i, 0)),
        pl.BlockSpec((BN, H), lambda i: (i, 0)),
        pl.BlockSpec((H, 2 * H), lambda i: (0, 0)),
        pl.BlockSpec((1, 2 * H), lambda i: (0, 0)),
        pl.BlockSpec((1, 2 * H), lambda i: (0, 0)),
        pl.BlockSpec((1, 2 * H), lambda i: (0, 0)),
        pl.BlockSpec((2 * H, H), lambda i: (0, 0)),
        pl.BlockSpec((1, H), lambda i: (0, 0)),
        pl.BlockSpec((1, H), lambda i: (0, 0)),
        pl.BlockSpec((1, H), lambda i: (0, 0)),
    ],
    out_specs=[
        pl.BlockSpec((BN, H), lambda i: (i, 0)),
        pl.BlockSpec((BN, 128), lambda i: (i, 0)),
        pl.BlockSpec((8, 128), lambda i: (0, 0)),
    ],
    out_shape=[
        jax.ShapeDtypeStruct((NP, H), jnp.float32),
        jax.ShapeDtypeStruct((NP, 128), jnp.float32),
        jax.ShapeDtypeStruct((8, 128), jnp.float32),
    ],
)


def _post_last_body(dnm_ref, hn_ref, hres_ref, w1_ref, b1_ref, g1_ref, bb1_ref,
                    w2_ref, b2_ref, lng_ref, lnb_ref, ow_ref, ob_ref,
                    out_ref, pool_ref):
    i = pl.program_id(0)
    p = dnm_ref[0] + dnm_ref[1]
    aggr = p[:, H:] / (p[:, :H] + 1e-16)
    u = jnp.dot(aggr + hn_ref[:, :H], w1_ref[...],
                preferred_element_type=jnp.float32) + b1_ref[...]
    m = jnp.mean(u, axis=-1, keepdims=True)
    v = jnp.mean((u - m) ** 2, axis=-1, keepdims=True)
    u = (u - m) / jnp.sqrt(v + 1e-5) * g1_ref[...] + bb1_ref[...]
    u = jnp.maximum(u, 0.0)
    hnew = jnp.dot(u, w2_ref[...],
                   preferred_element_type=jnp.float32) + b2_ref[...]
    hnew = hnew + hres_ref[...]
    m2 = jnp.mean(hnew, axis=-1, keepdims=True)
    v2 = jnp.mean((hnew - m2) ** 2, axis=-1, keepdims=True)
    hf = jnp.maximum(
        (hnew - m2) / jnp.sqrt(v2 + 1e-5) * lng_ref[...] + lnb_ref[...], 0.0)
    rows = lax.broadcasted_iota(jnp.int32, (BN, H), 0) + i * BN
    hf = jnp.where(rows < N, hf, 0.0)
    bm = jnp.max(hf, axis=0, keepdims=True)

    @pl.when(i == 0)
    def _():
        pool_ref[...] = bm

    @pl.when(i != 0)
    def _():
        pool_ref[...] = jnp.maximum(pool_ref[...], bm)

    @pl.when(i == GN - 1)
    def _():
        s = jnp.sum(pool_ref[...] * ow_ref[...]) + ob_ref[...]
        out_ref[...] = 1.0 / (1.0 + jnp.exp(-s))


_post_last = pl.pallas_call(
    _post_last_body,
    grid=(GN,),
    in_specs=[
        pl.BlockSpec((2, BN, 2 * H), lambda i: (0, i, 0)),
        pl.BlockSpec((BN, 128), lambda i: (i, 0)),
        pl.BlockSpec((BN, H), lambda i: (i, 0)),
        pl.BlockSpec((H, 2 * H), lambda i: (0, 0)),
        pl.BlockSpec((1, 2 * H), lambda i: (0, 0)),
        pl.BlockSpec((1, 2 * H), lambda i: (0, 0)),
        pl.BlockSpec((1, 2 * H), lambda i: (0, 0)),
        pl.BlockSpec((2 * H, H), lambda i: (0, 0)),
        pl.BlockSpec((1, H), lambda i: (0, 0)),
        pl.BlockSpec((1, H), lambda i: (0, 0)),
        pl.BlockSpec((1, H), lambda i: (0, 0)),
        pl.BlockSpec((1, H), lambda i: (0, 0)),
        pl.BlockSpec((1, 1), lambda i: (0, 0)),
    ],
    out_specs=pl.BlockSpec((1, 1), lambda i: (0, 0)),
    out_shape=jax.ShapeDtypeStruct((1, 1), jnp.float32),
    scratch_shapes=[pltpu.VMEM((1, H), jnp.float32)],
)


def kernel(x, edge_index, edge_attr, batch, node_W, node_b, e1_W, e1_b, e2_W,
           e2_b, ln_g, ln_b, mlp1_W, mlp1_b, mln_g, mln_b, mlp2_W, mlp2_b, t,
           out_W, out_b):
    f32 = jnp.float32
    xp = jnp.pad(x, ((0, NP - N), (0, 0)))
    src2 = jnp.concatenate(
        [edge_index[0], jnp.zeros((EP - E,), jnp.int32)]).reshape(CH_TOT, C)
    dst2 = jnp.concatenate(
        [edge_index[1], jnp.full((EP - E,), N, jnp.int32)]).reshape(CH_TOT, C)
    sd = jnp.stack([src2, dst2], axis=1)
    eap = jnp.pad(edge_attr, ((0, EP - E), (0, 0)))
    zr = jnp.zeros((RPT, 2 * H), f32)

    h0, mxh0 = _node_mm(xp, node_W, node_b.reshape(1, H))
    ea, mxe = _edge_mlp(eap, e1_W, e1_b.reshape(1, 32), e2_W,
                        e2_b.reshape(1, H))
    maxea = jnp.max(mxe)

    hn = h0
    hres = jnp.zeros((NP, H), f32)
    maxh = jnp.max(mxh0)
    for i in range(3):
        ti = t[i]
        msgmax = jnp.maximum(maxh + maxea, 0.0) + EPS
        Mi = jnp.where(ti >= 0, msgmax * ti, EPS * ti)
        tv = jnp.full((16,), ti, f32)
        mv = jnp.full((16,), Mi, f32)
        dnm = _sc_edge(hn, ea, sd, tv, mv, zr)
        if i < 2:
            hres, hn, mxh = _post_mid(
                dnm, hn, hres, mlp1_W[i], mlp1_b[i].reshape(1, 2 * H),
                mln_g[i].reshape(1, 2 * H), mln_b[i].reshape(1, 2 * H),
                mlp2_W[i], mlp2_b[i].reshape(1, H),
                ln_g[i + 1].reshape(1, H), ln_b[i + 1].reshape(1, H))
            maxh = jnp.max(mxh)
        else:
            out = _post_last(
                dnm, hn, hres, mlp1_W[i], mlp1_b[i].reshape(1, 2 * H),
                mln_g[i].reshape(1, 2 * H), mln_b[i].reshape(1, 2 * H),
                mlp2_W[i], mlp2_b[i].reshape(1, H),
                ln_g[0].reshape(1, H), ln_b[0].reshape(1, H),
                out_W.reshape(1, H), out_b.reshape(1, 1))
    return out
